# Initial kernel scaffold; baseline (speedup 1.0000x reference)
import jax
import jax.numpy as jnp
from jax.experimental import pallas as pl


def kernel(ref_nor, ref_abn, nor_keys, abn_keys, W_conv, b_conv, W_p, b_p, W_pm, b_pm, epoch, isTrain):
    raise NotImplementedError("write your pallas kernel here")


# speedup vs baseline: 4.0497x; 4.0497x over previous
import functools

import jax
import jax.numpy as jnp
from jax import lax
from jax.experimental import pallas as pl
from jax.experimental.pallas import tpu as pltpu, tpu_sc as plsc

_BS = 32
_N = 200
_D = 128
_F = 512
_K = 2048
_B = _BS * _N
_NW = 32
_BPW = _B // _NW


def _tc_body(x_ref, nk_ref, ak_ref, wc_ref, bc_ref, wp_ref, bp_ref,
             wpm1_ref, wpm2_ref, bpm_ref,
             rn_ref, p_ref, up_ref, ia_ref, ib_ref):
    x = x_ref[0]
    z0 = jnp.dot(x, wc_ref[0], preferred_element_type=jnp.float32)
    z1 = jnp.dot(x, wc_ref[1], preferred_element_type=jnp.float32)
    z2 = jnp.dot(x, wc_ref[2], preferred_element_type=jnp.float32)
    zrow = jnp.zeros((1, _D), jnp.float32)
    y = z1 + jnp.concatenate([zrow, z0[:-1]], axis=0) \
           + jnp.concatenate([z2[1:], zrow], axis=0)
    q = jnp.maximum(y + bc_ref[...], 0.0)
    rn_ref[0] = q

    nk = nk_ref[...]
    ak = ak_ref[...]
    dn = (((1,), (1,)), ((), ()))
    sn = lax.dot_general(q, nk, dn, preferred_element_type=jnp.float32)
    sa = lax.dot_general(q, ak, dn, preferred_element_type=jnp.float32)

    mn = jnp.max(sn, axis=1, keepdims=True)
    ma = jnp.max(sa, axis=1, keepdims=True)
    col = lax.broadcasted_iota(jnp.int32, (_N, _K), 1)
    ia = jnp.min(jnp.where(sn >= mn, col, _K), axis=1, keepdims=True)
    ib = jnp.min(jnp.where(sa >= ma, col, _K), axis=1, keepdims=True)

    m = jnp.maximum(mn, ma)
    pn = jnp.exp(sn - m)
    pa = jnp.exp(sa - m)
    l = jnp.sum(pn, axis=1, keepdims=True) + jnp.sum(pa, axis=1, keepdims=True)
    cm = (jnp.dot(pn, nk, preferred_element_type=jnp.float32)
          + jnp.dot(pa, ak, preferred_element_type=jnp.float32)) / l

    sp = jnp.sum(q * wp_ref[...], axis=1, keepdims=True) + bp_ref[...]
    p = 1.0 / (1.0 + jnp.exp(-sp))
    su = (jnp.sum(q * wpm1_ref[...], axis=1, keepdims=True)
          + jnp.sum(cm * wpm2_ref[...], axis=1, keepdims=True) + bpm_ref[...])
    up = 1.0 / (1.0 + jnp.exp(-su))

    p_ref[0] = p
    up_ref[0] = up
    ia_ref[0] = ia
    ib_ref[0] = ib


def _tc_call(x, nk, ak, wc, bc, wp, bp, wpm1, wpm2, bpm):
    const2 = lambda b: (0, 0)
    const3 = lambda b: (0, 0, 0)
    return pl.pallas_call(
        _tc_body,
        grid=(_BS,),
        in_specs=[
            pl.BlockSpec((1, _N, _F), lambda b: (b, 0, 0)),
            pl.BlockSpec((_K, _D), const2),
            pl.BlockSpec((_K, _D), const2),
            pl.BlockSpec((3, _F, _D), const3),
            pl.BlockSpec((1, _D), const2),
            pl.BlockSpec((1, _D), const2),
            pl.BlockSpec((1, 1), const2),
            pl.BlockSpec((1, _D), const2),
            pl.BlockSpec((1, _D), const2),
            pl.BlockSpec((1, 1), const2),
        ],
        out_specs=[
            pl.BlockSpec((1, _N, _D), lambda b: (b, 0, 0)),
            pl.BlockSpec((1, _N, 1), lambda b: (b, 0, 0)),
            pl.BlockSpec((1, _N, 1), lambda b: (b, 0, 0)),
            pl.BlockSpec((1, _N, 1), lambda b: (b, 0, 0)),
            pl.BlockSpec((1, _N, 1), lambda b: (b, 0, 0)),
        ],
        out_shape=[
            jax.ShapeDtypeStruct((_BS, _N, _D), jnp.float32),
            jax.ShapeDtypeStruct((_BS, _N, 1), jnp.float32),
            jax.ShapeDtypeStruct((_BS, _N, 1), jnp.float32),
            jax.ShapeDtypeStruct((_BS, _N, 1), jnp.int32),
            jax.ShapeDtypeStruct((_BS, _N, 1), jnp.int32),
        ],
        compiler_params=pltpu.CompilerParams(
            dimension_semantics=("arbitrary",),
        ),
    )(x, nk, ak, wc, bc, wp, bp, wpm1, wpm2, bpm)


def _sc_loss_body(q_hbm, nk_hbm, ak_hbm, ian_hbm, iab_hbm, outn_hbm, outa_hbm,
                  idxn_v, idxa_v, q_v, kn_v, ka_v, on_v, oa_v, sem):
    wid = lax.axis_index("s") * 2 + lax.axis_index("c")
    base = wid * _BPW
    pltpu.sync_copy(ian_hbm.at[pl.ds(base, _BPW)], idxn_v)
    pltpu.sync_copy(iab_hbm.at[pl.ds(base, _BPW)], idxa_v)
    pltpu.sync_copy(q_hbm.at[pl.ds(base, _BPW)], q_v)
    descs = []
    for off, sz in ((0, 104), (104, 96)):
        descs.append(pltpu.async_copy(
            nk_hbm.at[idxn_v.at[pl.ds(off, sz)]], kn_v.at[pl.ds(off, sz)], sem))
        descs.append(pltpu.async_copy(
            ak_hbm.at[idxa_v.at[pl.ds(off, sz)]], ka_v.at[pl.ds(off, sz)], sem))
    for d in descs:
        d.wait()

    lane = lax.broadcasted_iota(jnp.int32, (16,), 0)
    mask0 = lane == 0

    def lane_total(v):
        s = v
        for sh in (8, 4, 2, 1):
            s = s + s.at[(lane + sh) & 15].get(mode="promise_in_bounds")
        return s

    def row(i, carry):
        accn = jnp.zeros((16,), jnp.float32)
        acca = jnp.zeros((16,), jnp.float32)
        for c in range(_D // 16):
            sl = pl.ds(c * 16, 16)
            qv = q_v[i, sl]
            dnv = qv - kn_v[i, sl]
            dav = qv - ka_v[i, sl]
            accn = accn + dnv * dnv
            acca = acca + dav * dav
        tn = lane_total(accn) * (1.0 / _D)
        ta = lane_total(acca) * (1.0 / _D)
        iv = jnp.full((16,), i, jnp.int32)
        plsc.store_scatter(on_v, [iv], tn, mask=mask0)
        plsc.store_scatter(oa_v, [iv], ta, mask=mask0)
        return carry

    lax.fori_loop(0, _BPW, row, 0)
    pltpu.sync_copy(on_v, outn_hbm.at[pl.ds(base, _BPW)])
    pltpu.sync_copy(oa_v, outa_hbm.at[pl.ds(base, _BPW)])


@functools.cache
def _sc_loss_kernel():
    mesh = plsc.VectorSubcoreMesh(core_axis_name="c", subcore_axis_name="s")
    return pl.kernel(
        _sc_loss_body,
        out_type=(jax.ShapeDtypeStruct((_B,), jnp.float32),
                  jax.ShapeDtypeStruct((_B,), jnp.float32)),
        mesh=mesh,
        scratch_types=[
            pltpu.VMEM((_BPW,), jnp.int32),
            pltpu.VMEM((_BPW,), jnp.int32),
            pltpu.VMEM((_BPW, _D), jnp.float32),
            pltpu.VMEM((_BPW, _D), jnp.float32),
            pltpu.VMEM((_BPW, _D), jnp.float32),
            pltpu.VMEM((_BPW,), jnp.float32),
            pltpu.VMEM((_BPW,), jnp.float32),
            pltpu.SemaphoreType.DMA,
        ],
        compiler_params=pltpu.CompilerParams(needs_layout_passes=False),
    )


def kernel(ref_nor, ref_abn, nor_keys, abn_keys, W_conv, b_conv, W_p, b_p,
           W_pm, b_pm, epoch, isTrain):
    wc = jnp.transpose(W_conv, (2, 1, 0))
    bc = b_conv.reshape(1, _D)
    wp = W_p.reshape(1, _D)
    bp = b_p.reshape(1, 1)
    wpm1 = W_pm[:, :_D].reshape(1, _D)
    wpm2 = W_pm[:, _D:].reshape(1, _D)
    bpm = b_pm.reshape(1, 1)

    rn, p, up, ia, ib = _tc_call(ref_nor, nor_keys, abn_keys, wc, bc,
                                 wp, bp, wpm1, wpm2, bpm)

    q = rn.reshape(_B, _D)
    ln, la = _sc_loss_kernel()(q, nor_keys, abn_keys,
                               ia.reshape(_B), ib.reshape(_B))

    return (p.reshape(_BS, _N), up.reshape(_BS, _N), ln, la, rn)


# speedup vs baseline: 4.3832x; 1.0824x over previous
import functools

import jax
import jax.numpy as jnp
from jax import lax
from jax.experimental import pallas as pl
from jax.experimental.pallas import tpu as pltpu, tpu_sc as plsc

_BS = 32
_N = 200
_D = 128
_F = 512
_K = 2048
_B = _BS * _N
_NW = 32
_BPW = _B // _NW


def _tc_body(x_ref, nk_ref, ak_ref, wc_ref, bc_ref, wp_ref, bp_ref,
             wpm1_ref, wpm2_ref, bpm_ref,
             rn_ref, p_ref, up_ref, ia_ref, ib_ref):
    x = x_ref[0]
    z0 = jnp.dot(x, wc_ref[0], preferred_element_type=jnp.float32)
    z1 = jnp.dot(x, wc_ref[1], preferred_element_type=jnp.float32)
    z2 = jnp.dot(x, wc_ref[2], preferred_element_type=jnp.float32)
    zrow = jnp.zeros((1, _D), jnp.float32)
    y = z1 + jnp.concatenate([zrow, z0[:-1]], axis=0) \
           + jnp.concatenate([z2[1:], zrow], axis=0)
    q = jnp.maximum(y + bc_ref[...], 0.0)
    rn_ref[0] = q

    nk = nk_ref[...]
    ak = ak_ref[...]
    dn = (((1,), (1,)), ((), ()))
    sn = lax.dot_general(q, nk, dn, preferred_element_type=jnp.float32)
    sa = lax.dot_general(q, ak, dn, preferred_element_type=jnp.float32)

    mn = jnp.max(sn, axis=1, keepdims=True)
    ma = jnp.max(sa, axis=1, keepdims=True)
    maskn = jnp.where(sn >= mn, 1.0, 0.0).astype(jnp.bfloat16)
    maska = jnp.where(sa >= ma, 1.0, 0.0).astype(jnp.bfloat16)
    coli = lax.broadcasted_iota(jnp.int32, (_K, 1), 0)
    colhl = jnp.concatenate(
        [(coli >> 7).astype(jnp.bfloat16), (coli & 127).astype(jnp.bfloat16)],
        axis=1)
    rn_idx = jnp.dot(maskn, colhl, preferred_element_type=jnp.float32)
    ra_idx = jnp.dot(maska, colhl, preferred_element_type=jnp.float32)
    iaf = 128.0 * rn_idx[:, 0:1] + rn_idx[:, 1:2]
    ibf = 128.0 * ra_idx[:, 0:1] + ra_idx[:, 1:2]
    ia = jnp.clip(iaf, 0.0, float(_K - 1)).astype(jnp.int32)
    ib = jnp.clip(ibf, 0.0, float(_K - 1)).astype(jnp.int32)

    m = jnp.maximum(mn, ma)
    pn = jnp.exp(sn - m)
    pa = jnp.exp(sa - m)
    l = jnp.sum(pn, axis=1, keepdims=True) + jnp.sum(pa, axis=1, keepdims=True)
    pnb = pn.astype(jnp.bfloat16)
    pab = pa.astype(jnp.bfloat16)
    nkb = nk.astype(jnp.bfloat16)
    akb = ak.astype(jnp.bfloat16)
    cm = (jnp.dot(pnb, nkb, preferred_element_type=jnp.float32)
          + jnp.dot(pab, akb, preferred_element_type=jnp.float32)) / l

    sp = jnp.sum(q * wp_ref[...], axis=1, keepdims=True) + bp_ref[...]
    p = 1.0 / (1.0 + jnp.exp(-sp))
    su = (jnp.sum(q * wpm1_ref[...], axis=1, keepdims=True)
          + jnp.sum(cm * wpm2_ref[...], axis=1, keepdims=True) + bpm_ref[...])
    up = 1.0 / (1.0 + jnp.exp(-su))

    p_ref[0] = p
    up_ref[0] = up
    ia_ref[0] = ia
    ib_ref[0] = ib


def _tc_call(x, nk, ak, wc, bc, wp, bp, wpm1, wpm2, bpm):
    const2 = lambda b: (0, 0)
    const3 = lambda b: (0, 0, 0)
    return pl.pallas_call(
        _tc_body,
        grid=(_BS,),
        in_specs=[
            pl.BlockSpec((1, _N, _F), lambda b: (b, 0, 0)),
            pl.BlockSpec((_K, _D), const2),
            pl.BlockSpec((_K, _D), const2),
            pl.BlockSpec((3, _F, _D), const3),
            pl.BlockSpec((1, _D), const2),
            pl.BlockSpec((1, _D), const2),
            pl.BlockSpec((1, 1), const2),
            pl.BlockSpec((1, _D), const2),
            pl.BlockSpec((1, _D), const2),
            pl.BlockSpec((1, 1), const2),
        ],
        out_specs=[
            pl.BlockSpec((1, _N, _D), lambda b: (b, 0, 0)),
            pl.BlockSpec((1, _N, 1), lambda b: (b, 0, 0)),
            pl.BlockSpec((1, _N, 1), lambda b: (b, 0, 0)),
            pl.BlockSpec((1, _N, 1), lambda b: (b, 0, 0)),
            pl.BlockSpec((1, _N, 1), lambda b: (b, 0, 0)),
        ],
        out_shape=[
            jax.ShapeDtypeStruct((_BS, _N, _D), jnp.float32),
            jax.ShapeDtypeStruct((_BS, _N, 1), jnp.float32),
            jax.ShapeDtypeStruct((_BS, _N, 1), jnp.float32),
            jax.ShapeDtypeStruct((_BS, _N, 1), jnp.int32),
            jax.ShapeDtypeStruct((_BS, _N, 1), jnp.int32),
        ],
        compiler_params=pltpu.CompilerParams(
            dimension_semantics=("arbitrary",),
        ),
    )(x, nk, ak, wc, bc, wp, bp, wpm1, wpm2, bpm)


def _sc_loss_body(q_hbm, nk_hbm, ak_hbm, ian_hbm, iab_hbm, outn_hbm, outa_hbm,
                  idxn_v, idxa_v, q_v, kn_v, ka_v, on_v, oa_v, sem):
    wid = lax.axis_index("s") * 2 + lax.axis_index("c")
    base = wid * _BPW
    pltpu.sync_copy(ian_hbm.at[pl.ds(base, _BPW)], idxn_v)
    pltpu.sync_copy(iab_hbm.at[pl.ds(base, _BPW)], idxa_v)
    pltpu.sync_copy(q_hbm.at[pl.ds(base, _BPW)], q_v)
    descs = []
    for off, sz in ((0, 104), (104, 96)):
        descs.append(pltpu.async_copy(
            nk_hbm.at[idxn_v.at[pl.ds(off, sz)]], kn_v.at[pl.ds(off, sz)], sem))
        descs.append(pltpu.async_copy(
            ak_hbm.at[idxa_v.at[pl.ds(off, sz)]], ka_v.at[pl.ds(off, sz)], sem))
    for d in descs:
        d.wait()

    lane = lax.broadcasted_iota(jnp.int32, (16,), 0)
    mask0 = lane == 0

    def lane_total(v):
        s = v
        for sh in (8, 4, 2, 1):
            s = s + s.at[(lane + sh) & 15].get(mode="promise_in_bounds")
        return s

    def row(i, carry):
        accn = jnp.zeros((16,), jnp.float32)
        acca = jnp.zeros((16,), jnp.float32)
        for c in range(_D // 16):
            sl = pl.ds(c * 16, 16)
            qv = q_v[i, sl]
            dnv = qv - kn_v[i, sl]
            dav = qv - ka_v[i, sl]
            accn = accn + dnv * dnv
            acca = acca + dav * dav
        tn = lane_total(accn) * (1.0 / _D)
        ta = lane_total(acca) * (1.0 / _D)
        iv = jnp.full((16,), i, jnp.int32)
        plsc.store_scatter(on_v, [iv], tn, mask=mask0)
        plsc.store_scatter(oa_v, [iv], ta, mask=mask0)
        return carry

    lax.fori_loop(0, _BPW, row, 0)
    pltpu.sync_copy(on_v, outn_hbm.at[pl.ds(base, _BPW)])
    pltpu.sync_copy(oa_v, outa_hbm.at[pl.ds(base, _BPW)])


@functools.cache
def _sc_loss_kernel():
    mesh = plsc.VectorSubcoreMesh(core_axis_name="c", subcore_axis_name="s")
    return pl.kernel(
        _sc_loss_body,
        out_type=(jax.ShapeDtypeStruct((_B,), jnp.float32),
                  jax.ShapeDtypeStruct((_B,), jnp.float32)),
        mesh=mesh,
        scratch_types=[
            pltpu.VMEM((_BPW,), jnp.int32),
            pltpu.VMEM((_BPW,), jnp.int32),
            pltpu.VMEM((_BPW, _D), jnp.float32),
            pltpu.VMEM((_BPW, _D), jnp.float32),
            pltpu.VMEM((_BPW, _D), jnp.float32),
            pltpu.VMEM((_BPW,), jnp.float32),
            pltpu.VMEM((_BPW,), jnp.float32),
            pltpu.SemaphoreType.DMA,
        ],
        compiler_params=pltpu.CompilerParams(needs_layout_passes=False),
    )


def kernel(ref_nor, ref_abn, nor_keys, abn_keys, W_conv, b_conv, W_p, b_p,
           W_pm, b_pm, epoch, isTrain):
    wc = jnp.transpose(W_conv, (2, 1, 0))
    bc = b_conv.reshape(1, _D)
    wp = W_p.reshape(1, _D)
    bp = b_p.reshape(1, 1)
    wpm1 = W_pm[:, :_D].reshape(1, _D)
    wpm2 = W_pm[:, _D:].reshape(1, _D)
    bpm = b_pm.reshape(1, 1)

    rn, p, up, ia, ib = _tc_call(ref_nor, nor_keys, abn_keys, wc, bc,
                                 wp, bp, wpm1, wpm2, bpm)

    q = rn.reshape(_B, _D)
    ln, la = _sc_loss_kernel()(q, nor_keys, abn_keys,
                               ia.reshape(_B), ib.reshape(_B))

    return (p.reshape(_BS, _N), up.reshape(_BS, _N), ln, la, rn)
